# baseline (speedup 1.0000x reference)
import functools

import jax
import jax.numpy as jnp
from jax import lax
from jax.experimental import pallas as pl
from jax.experimental.pallas import tpu as pltpu
from jax.experimental.pallas import tpu_sc as plsc

N = 10000
NPAD = 10112
E = 320000
NC = 2
NS = 16
EPS = E // NS
K = 40
NCHUNK = EPS // K
ROWS_PS = NPAD // NS

_LANE = 16


def _vgather16(x, idx):
  dn = lax.GatherDimensionNumbers(
      offset_dims=(), collapsed_slice_dims=(0,), start_index_map=(0,))
  return lax.gather(x, idx[:, None], dn, (1,),
                    mode=lax.GatherScatterMode.PROMISE_IN_BOUNDS)


def _edge_pipeline(gi2d, si2d, g_hbm, d_hbm, out_hbm, sid,
                   gbuf, dbuf, mbuf, ibs, ibd, sems, acc, compute, mw,
                   kc, nchunk):
  isem = sems[0:4]
  gsem = sems[4:6]
  dsem = sems[6:8]
  ssem = sems[8:10]

  zero16 = jnp.zeros((_LANE,), jnp.float32)

  def zrow(r, _):
    for v in range(mw // _LANE):
      mbuf[0, r, pl.ds(v * _LANE, _LANE)] = zero16
    return 0
  lax.fori_loop(0, kc, zrow, 0)
  nz = ROWS_PS // kc
  for z in range(nz):
    pltpu.sync_copy(mbuf.at[0], acc.at[pl.ds(sid * ROWS_PS + z * kc, kc)])
  rem = ROWS_PS - nz * kc
  if rem:
    pltpu.sync_copy(mbuf.at[0, pl.ds(0, rem)],
                    acc.at[pl.ds(sid * ROWS_PS + nz * kc, rem)])
  plsc.subcore_barrier()

  row0 = sid * nchunk

  def issue_idx(slot, crow):
    pltpu.async_copy(gi2d.at[pl.ds(crow, 1)], ibs.at[pl.ds(slot, 1)],
                     isem[slot])
    pltpu.async_copy(si2d.at[pl.ds(crow, 1)], ibd.at[pl.ds(slot, 1)],
                     isem[slot])

  def wait_idx(slot, crow):
    pltpu.make_async_copy(gi2d.at[pl.ds(crow, 1)], ibs.at[pl.ds(slot, 1)],
                          isem[slot]).wait()
    pltpu.make_async_copy(si2d.at[pl.ds(crow, 1)], ibd.at[pl.ds(slot, 1)],
                          isem[slot]).wait()

  def issue_gather(slot, b):
    pltpu.async_copy(g_hbm.at[ibs.at[slot]], gbuf.at[b], gsem[b])
    pltpu.async_copy(d_hbm.at[ibd.at[slot]], dbuf.at[b], dsem[b])

  def wait_gather(slot, b):
    pltpu.make_async_copy(g_hbm.at[ibs.at[slot]], gbuf.at[b],
                          gsem[b]).wait()
    pltpu.make_async_copy(d_hbm.at[ibd.at[slot]], dbuf.at[b],
                          dsem[b]).wait()

  def issue_scatter(slot, b):
    pltpu.async_copy(mbuf.at[b], acc.at[ibd.at[slot]], ssem[b], add=True)

  def wait_scatter(slot, b):
    pltpu.make_async_copy(mbuf.at[b], acc.at[ibd.at[slot]],
                          ssem[b]).wait()

  pltpu.sync_copy(gi2d.at[pl.ds(row0, 1)], ibs.at[pl.ds(0, 1)])
  pltpu.sync_copy(si2d.at[pl.ds(row0, 1)], ibd.at[pl.ds(0, 1)])
  pltpu.sync_copy(gi2d.at[pl.ds(row0 + 1, 1)], ibs.at[pl.ds(1, 1)])
  pltpu.sync_copy(si2d.at[pl.ds(row0 + 1, 1)], ibd.at[pl.ds(1, 1)])
  issue_gather(0, 0)

  def body(j, _):
    for u in range(4):
      b = u % 2
      c = row0 + 4 * j + u

      if u < 2:
        @pl.when(j > 0)
        def _():
          wait_scatter((u + 2) % 4, b)
      else:
        wait_scatter(u - 2, b)

      if u < 2:
        issue_idx((u + 2) % 4, c + 2)
      else:
        @pl.when(j < nchunk // 4 - 1)
        def _():
          issue_idx((u + 2) % 4, c + 2)

      if u == 0:
        @pl.when(j > 0)
        def _():
          wait_idx((u + 1) % 4, c + 1)
        issue_gather((u + 1) % 4, 1 - b)
      elif u == 3:
        @pl.when(j < nchunk // 4 - 1)
        def _():
          wait_idx((u + 1) % 4, c + 1)
          issue_gather((u + 1) % 4, 1 - b)
      else:
        wait_idx((u + 1) % 4, c + 1)
        issue_gather((u + 1) % 4, 1 - b)

      wait_gather(u, b)
      compute(gbuf, dbuf, mbuf, b)
      issue_scatter(u, b)
    return 0

  lax.fori_loop(0, nchunk // 4, body, 0)
  wait_scatter(2, 0)
  wait_scatter(3, 1)
  plsc.subcore_barrier()
  pltpu.sync_copy(acc.at[pl.ds(sid * ROWS_PS, ROWS_PS)],
                  out_hbm.at[pl.ds(sid * ROWS_PS, ROWS_PS)])


def _compute1(gbuf, dbuf, mbuf, b, kc):
  iota16 = lax.iota(jnp.int32, _LANE)
  pats = []
  for v in range(9):
    h0 = (16 * v) // 12
    t = (h0 + 1) * 12 - 16 * v
    pats.append(jnp.where(iota16 >= t, jnp.int32(h0 + 1), jnp.int32(h0)))
  himask = jnp.int32(-65536)

  def edge(k, _):
    u4 = gbuf[b, k, pl.ds(64, _LANE)]
    lo4 = plsc.bitcast(u4 << 16, jnp.float32)
    hi4 = plsc.bitcast(u4 & himask, jnp.float32)
    e = hi4 + dbuf[b, k, :]
    e = jnp.maximum(e, 0.2 * e)
    ex = jnp.exp(e)
    mbuf[b, k, pl.ds(144, _LANE)] = ex
    mbuf[b, k, pl.ds(128, _LANE)] = lo4 * _vgather16(ex, pats[8])
    for v in range(4):
      u = gbuf[b, k, pl.ds(16 * v, _LANE)]
      lo = plsc.bitcast(u << 16, jnp.float32)
      hi = plsc.bitcast(u & himask, jnp.float32)
      mbuf[b, k, pl.ds(32 * v, _LANE)] = lo * _vgather16(ex, pats[2 * v])
      mbuf[b, k, pl.ds(32 * v + 16, _LANE)] = \
          hi * _vgather16(ex, pats[2 * v + 1])
    return 0
  lax.fori_loop(0, kc, edge, 0)


def _compute2(gbuf, dbuf, mbuf, b, kc):
  iota16 = lax.iota(jnp.int32, _LANE)
  is15 = iota16 == 15

  def edge(k, _):
    g0 = gbuf[b, k, pl.ds(0, _LANE)]
    als = gbuf[b, k, pl.ds(_LANE, _LANE)]
    e = als + dbuf[b, k, :]
    e = jnp.maximum(e, 0.2 * e)
    ex = jnp.exp(e)
    mbuf[b, k, :] = jnp.where(is15, ex, g0 * ex)
    return 0
  lax.fori_loop(0, kc, edge, 0)


def _make_sc_layer(gw, gdtype, mw, compute, kc):
  nchunk = EPS // kc

  def body(ei0_2d, ei1_2d, g_in, d_in, g_out, d_out, out_in, out_out,
           gbuf, dbuf, mbuf, ibs, ibd, *rest):
    sems = rest[:10]
    acc = rest[10]
    cid = lax.axis_index("c")
    sid = lax.axis_index("s")

    cmp = functools.partial(compute, kc=kc)

    @pl.when(cid == 0)
    def _():
      _edge_pipeline(ei0_2d, ei1_2d, g_in, d_in, out_in, sid,
                     gbuf, dbuf, mbuf, ibs, ibd, sems, acc,
                     cmp, mw, kc, nchunk)

    @pl.when(cid == 1)
    def _():
      _edge_pipeline(ei1_2d, ei0_2d, g_out, d_out, out_out, sid,
                     gbuf, dbuf, mbuf, ibs, ibd, sems, acc,
                     cmp, mw, kc, nchunk)

  def call(ei0_2d, ei1_2d, g_in, d_in, g_out, d_out):
    return pl.kernel(
        body,
        out_type=(jax.ShapeDtypeStruct((NPAD, mw), jnp.float32),
                  jax.ShapeDtypeStruct((NPAD, mw), jnp.float32)),
        mesh=plsc.VectorSubcoreMesh(core_axis_name="c",
                                    subcore_axis_name="s"),
        compiler_params=pltpu.CompilerParams(use_tc_tiling_on_sc=False, needs_layout_passes=False),
        scratch_types=[
            pltpu.VMEM((2, kc, gw), gdtype),
            pltpu.VMEM((2, kc, _LANE), jnp.float32),
            pltpu.VMEM((2, kc, mw), jnp.float32),
            pltpu.VMEM((4, kc), jnp.int32),
            pltpu.VMEM((4, kc), jnp.int32),
        ] + [pltpu.SemaphoreType.DMA] * 10
          + [pltpu.VMEM_SHARED((NPAD, mw), jnp.float32)],
    )(ei0_2d, ei1_2d, g_in, d_in, g_out, d_out)

  return call


K2 = 200
_sc_layer1 = _make_sc_layer(80, jnp.int32, 160, _compute1, K)
_sc_layer2 = _make_sc_layer(32, jnp.float32, _LANE, _compute2, K2)


_BN = 400
_GRID = N // _BN


def _mm_body(x_ref, *refs):
  nw = len(refs) // 2
  xb = x_ref[...]
  for i in range(nw):
    y = jnp.dot(xb, refs[i][...], preferred_element_type=jnp.float32)
    refs[nw + i][...] = y.astype(refs[nw + i].dtype)


def _mm(x, ws, dtypes=None):
  din = x.shape[1]
  if dtypes is None:
    dtypes = [jnp.float32] * len(ws)
  in_specs = [pl.BlockSpec((_BN, din), lambda i: (i, 0))]
  in_specs += [pl.BlockSpec(w.shape, lambda i: (0, 0)) for w in ws]
  return pl.pallas_call(
      _mm_body,
      grid=(_GRID,),
      in_specs=in_specs,
      out_specs=[pl.BlockSpec((_BN, w.shape[1]), lambda i: (i, 0))
                 for w in ws],
      out_shape=[jax.ShapeDtypeStruct((N, w.shape[1]), dt)
                 for w, dt in zip(ws, dtypes)],
  )(x, *ws)


def _combine1_body(ai_ref, ao_ref, xl_ref, bi_ref, bo_ref, bl_ref, o_ref):
  r = lax.broadcasted_iota(jnp.int32, (12, 144), 0)
  c = lax.broadcasted_iota(jnp.int32, (12, 144), 1) // 12
  mexp = (r == c).astype(jnp.float32)

  def branch(a_ref, b_ref):
    a = a_ref[...]
    num = a[:, :144]
    den = a[:, 144:156]
    inv = 1.0 / (den + 1e-16)
    return num * jnp.dot(inv, mexp, preferred_element_type=jnp.float32) \
        + b_ref[...]

  xi = branch(ai_ref, bi_ref)
  xo = branch(ao_ref, bo_ref)
  h = 0.5 * xi + 0.5 * xo + xl_ref[...] + bl_ref[...]
  o_ref[...] = jnp.maximum(h, 0.0)


def _combine1(ai, ao, xl, bi, bo, bl):
  return pl.pallas_call(
      _combine1_body,
      grid=(_GRID,),
      in_specs=[
          pl.BlockSpec((_BN, 160), lambda i: (i, 0)),
          pl.BlockSpec((_BN, 160), lambda i: (i, 0)),
          pl.BlockSpec((_BN, 144), lambda i: (i, 0)),
          pl.BlockSpec((1, 144), lambda i: (0, 0)),
          pl.BlockSpec((1, 144), lambda i: (0, 0)),
          pl.BlockSpec((1, 144), lambda i: (0, 0)),
      ],
      out_specs=pl.BlockSpec((_BN, 144), lambda i: (i, 0)),
      out_shape=jax.ShapeDtypeStruct((N, 144), jnp.float32),
  )(ai, ao, xl, bi, bo, bl)


def _combine2_body(ai_ref, ao_ref, xl_ref, bi_ref, bo_ref, bl_ref, o_ref):
  def branch(a_ref, b_ref):
    a = a_ref[...]
    num = a[:, :10]
    den = a[:, 15:16]
    inv = 1.0 / (den + 1e-16)
    return num * inv + b_ref[...]

  xi = branch(ai_ref, bi_ref)
  xo = branch(ao_ref, bo_ref)
  o_ref[...] = 0.5 * xi + 0.5 * xo + xl_ref[...] + bl_ref[...]


def _combine2(ai, ao, xl, bi, bo, bl):
  return pl.pallas_call(
      _combine2_body,
      grid=(_GRID,),
      in_specs=[
          pl.BlockSpec((_BN, _LANE), lambda i: (i, 0)),
          pl.BlockSpec((_BN, _LANE), lambda i: (i, 0)),
          pl.BlockSpec((_BN, 10), lambda i: (i, 0)),
          pl.BlockSpec((1, 10), lambda i: (0, 0)),
          pl.BlockSpec((1, 10), lambda i: (0, 0)),
          pl.BlockSpec((1, 10), lambda i: (0, 0)),
      ],
      out_specs=pl.BlockSpec((_BN, 10), lambda i: (i, 0)),
      out_shape=jax.ShapeDtypeStruct((N, 10), jnp.float32),
  )(ai, ao, xl, bi, bo, bl)


def _fold(w, a):
  h, c = a.shape
  return jnp.einsum("dhc,hc->dh", w.reshape(w.shape[0], h, c), a)


def kernel(x, edge_index, w1i, a1si, a1di, b1i, w1o, a1so, a1do, b1o,
           lin1w, lin1b, w2i, a2si, a2di, b2i, w2o, a2so, a2do, b2o,
           lin2w, lin2b):
  ei0 = edge_index[0].reshape(E // K, K)
  ei1 = edge_index[1].reshape(E // K, K)
  ei0b = edge_index[0].reshape(E // K2, K2)
  ei1b = edge_index[1].reshape(E // K2, K2)

  perm = []
  for w in range(80):
    v, j = w // 16, w % 16
    perm.extend([32 * v + j, 32 * v + 16 + j])

  z4 = jnp.zeros((128, 4), jnp.float32)
  wg_in = jnp.concatenate([w1i, _fold(w1i, a1si), z4], axis=1)[:, perm]
  wg_out = jnp.concatenate([w1o, _fold(w1o, a1so), z4], axis=1)[:, perm]
  wd_in = jnp.concatenate([_fold(w1i, a1di), z4], axis=1)
  wd_out = jnp.concatenate([_fold(w1o, a1do), z4], axis=1)

  g_in, g_out, d_in, d_out, xl1 = _mm(
      x, [wg_in, wg_out, wd_in, wd_out, lin1w],
      dtypes=[jnp.bfloat16, jnp.bfloat16, jnp.float32, jnp.float32,
              jnp.float32])

  def pack32(g):
    return lax.bitcast_convert_type(g.reshape(N, 80, 2), jnp.int32)

  acc_i, acc_o = _sc_layer1(ei0, ei1, pack32(g_in), d_in,
                            pack32(g_out), d_out)

  h1 = _combine1(acc_i, acc_o, xl1,
                 b1i.reshape(1, 144), b1o.reshape(1, 144),
                 lin1b.reshape(1, 144))

  ones16 = jnp.ones((1, _LANE), jnp.float32)
  z6 = jnp.zeros((144, 6), jnp.float32)
  wg2_in = jnp.concatenate([w2i, z6, _fold(w2i, a2si) @ ones16], axis=1)
  wg2_out = jnp.concatenate([w2o, z6, _fold(w2o, a2so) @ ones16], axis=1)
  wd2_in = _fold(w2i, a2di) @ ones16
  wd2_out = _fold(w2o, a2do) @ ones16

  g2_in, g2_out, d2_in, d2_out, xl2 = _mm(
      h1, [wg2_in, wg2_out, wd2_in, wd2_out, lin2w])

  acc2_i, acc2_o = _sc_layer2(ei0b, ei1b, g2_in, d2_in, g2_out, d2_out)

  return _combine2(acc2_i, acc2_o, xl2,
                   b2i.reshape(1, 10), b2o.reshape(1, 10),
                   lin2b.reshape(1, 10))


# speedup vs baseline: 1.4725x; 1.4725x over previous
import functools

import jax
import jax.numpy as jnp
from jax import lax
from jax.experimental import pallas as pl
from jax.experimental.pallas import tpu as pltpu
from jax.experimental.pallas import tpu_sc as plsc

N = 10000
NPAD = 10112
E = 320000
NC = 2
NS = 16
EPS = E // NS
K = 40
NCHUNK = EPS // K
ROWS_PS = NPAD // NS

_LANE = 16


def _vgather16(x, idx):
  dn = lax.GatherDimensionNumbers(
      offset_dims=(), collapsed_slice_dims=(0,), start_index_map=(0,))
  return lax.gather(x, idx[:, None], dn, (1,),
                    mode=lax.GatherScatterMode.PROMISE_IN_BOUNDS)


def _edge_pipeline(gi2d, si2d, g_hbm, d_hbm, out_hbm, sid,
                   gbuf, dbuf, mbuf, ibs, ibd, sems, acc, compute, mw,
                   kc, nchunk):
  isem = sems[0:4]
  gsem = sems[4:6]
  dsem = sems[6:8]
  ssem = sems[8:10]

  zero16 = jnp.zeros((_LANE,), jnp.float32)

  def zrow(r, _):
    for v in range(mw // _LANE):
      mbuf[0, r, pl.ds(v * _LANE, _LANE)] = zero16
    return 0
  lax.fori_loop(0, kc, zrow, 0)
  nz = ROWS_PS // kc
  for z in range(nz):
    pltpu.sync_copy(mbuf.at[0], acc.at[pl.ds(sid * ROWS_PS + z * kc, kc)])
  rem = ROWS_PS - nz * kc
  if rem:
    pltpu.sync_copy(mbuf.at[0, pl.ds(0, rem)],
                    acc.at[pl.ds(sid * ROWS_PS + nz * kc, rem)])
  plsc.subcore_barrier()

  row0 = sid * nchunk

  def issue_idx(slot, crow):
    pltpu.async_copy(gi2d.at[pl.ds(crow, 1)], ibs.at[pl.ds(slot, 1)],
                     isem[slot])
    pltpu.async_copy(si2d.at[pl.ds(crow, 1)], ibd.at[pl.ds(slot, 1)],
                     isem[slot])

  def wait_idx(slot, crow):
    pltpu.make_async_copy(gi2d.at[pl.ds(crow, 1)], ibs.at[pl.ds(slot, 1)],
                          isem[slot]).wait()
    pltpu.make_async_copy(si2d.at[pl.ds(crow, 1)], ibd.at[pl.ds(slot, 1)],
                          isem[slot]).wait()

  def issue_gather(slot, b):
    pltpu.async_copy(g_hbm.at[ibs.at[slot]], gbuf.at[b], gsem[b])
    pltpu.async_copy(d_hbm.at[ibd.at[slot]], dbuf.at[b], dsem[b])

  def wait_gather(slot, b):
    pltpu.make_async_copy(g_hbm.at[ibs.at[slot]], gbuf.at[b],
                          gsem[b]).wait()
    pltpu.make_async_copy(d_hbm.at[ibd.at[slot]], dbuf.at[b],
                          dsem[b]).wait()

  def issue_scatter(slot, b):
    pltpu.async_copy(mbuf.at[b], acc.at[ibd.at[slot]], ssem[b], add=True)

  def wait_scatter(slot, b):
    pltpu.make_async_copy(mbuf.at[b], acc.at[ibd.at[slot]],
                          ssem[b]).wait()

  pltpu.sync_copy(gi2d.at[pl.ds(row0, 1)], ibs.at[pl.ds(0, 1)])
  pltpu.sync_copy(si2d.at[pl.ds(row0, 1)], ibd.at[pl.ds(0, 1)])
  pltpu.sync_copy(gi2d.at[pl.ds(row0 + 1, 1)], ibs.at[pl.ds(1, 1)])
  pltpu.sync_copy(si2d.at[pl.ds(row0 + 1, 1)], ibd.at[pl.ds(1, 1)])
  issue_gather(0, 0)

  def body(j, _):
    for u in range(4):
      b = u % 2
      c = row0 + 4 * j + u

      if u < 2:
        @pl.when(j > 0)
        def _():
          wait_scatter((u + 2) % 4, b)
      else:
        wait_scatter(u - 2, b)

      if u < 2:
        issue_idx((u + 2) % 4, c + 2)
      else:
        @pl.when(j < nchunk // 4 - 1)
        def _():
          issue_idx((u + 2) % 4, c + 2)

      if u == 0:
        @pl.when(j > 0)
        def _():
          wait_idx((u + 1) % 4, c + 1)
        issue_gather((u + 1) % 4, 1 - b)
      elif u == 3:
        @pl.when(j < nchunk // 4 - 1)
        def _():
          wait_idx((u + 1) % 4, c + 1)
          issue_gather((u + 1) % 4, 1 - b)
      else:
        wait_idx((u + 1) % 4, c + 1)
        issue_gather((u + 1) % 4, 1 - b)

      wait_gather(u, b)
      compute(gbuf, dbuf, mbuf, b)
      issue_scatter(u, b)
    return 0

  lax.fori_loop(0, nchunk // 4, body, 0)
  wait_scatter(2, 0)
  wait_scatter(3, 1)
  plsc.subcore_barrier()
  pltpu.sync_copy(acc.at[pl.ds(sid * ROWS_PS, ROWS_PS)],
                  out_hbm.at[pl.ds(sid * ROWS_PS, ROWS_PS)])


def _compute1(gbuf, dbuf, mbuf, b, kc):
  iota16 = lax.iota(jnp.int32, _LANE)
  pats = []
  for v in range(9):
    h0 = (16 * v) // 12
    t = (h0 + 1) * 12 - 16 * v
    pats.append(jnp.where(iota16 >= t, jnp.int32(h0 + 1), jnp.int32(h0)))

  def edge2(k2, _):
    for s in range(2):
      k = 2 * k2 + s
      a = gbuf[b, k, pl.ds(144, _LANE)]
      d = dbuf[b, k, :]
      e = a + d
      e = jnp.maximum(e, 0.2 * e)
      ex = jnp.exp(e)
      mbuf[b, k, pl.ds(144, _LANE)] = ex
      for v in range(9):
        m = _vgather16(ex, pats[v])
        mbuf[b, k, pl.ds(16 * v, _LANE)] = \
            gbuf[b, k, pl.ds(16 * v, _LANE)] * m
    return 0
  lax.fori_loop(0, kc // 2, edge2, 0)


def _compute2(gbuf, dbuf, mbuf, b, kc):
  iota16 = lax.iota(jnp.int32, _LANE)
  is15 = iota16 == 15

  def edge(k, _):
    g0 = gbuf[b, k, pl.ds(0, _LANE)]
    als = gbuf[b, k, pl.ds(_LANE, _LANE)]
    e = als + dbuf[b, k, :]
    e = jnp.maximum(e, 0.2 * e)
    ex = jnp.exp(e)
    mbuf[b, k, :] = jnp.where(is15, ex, g0 * ex)
    return 0
  lax.fori_loop(0, kc, edge, 0)


def _make_sc_layer(gw, gdtype, mw, compute, kc):
  nchunk = EPS // kc

  def body(ei0_2d, ei1_2d, g_in, d_in, g_out, d_out, out_in, out_out,
           gbuf, dbuf, mbuf, ibs, ibd, *rest):
    sems = rest[:10]
    acc = rest[10]
    cid = lax.axis_index("c")
    sid = lax.axis_index("s")

    cmp = functools.partial(compute, kc=kc)

    @pl.when(cid == 0)
    def _():
      _edge_pipeline(ei0_2d, ei1_2d, g_in, d_in, out_in, sid,
                     gbuf, dbuf, mbuf, ibs, ibd, sems, acc,
                     cmp, mw, kc, nchunk)

    @pl.when(cid == 1)
    def _():
      _edge_pipeline(ei1_2d, ei0_2d, g_out, d_out, out_out, sid,
                     gbuf, dbuf, mbuf, ibs, ibd, sems, acc,
                     cmp, mw, kc, nchunk)

  def call(ei0_2d, ei1_2d, g_in, d_in, g_out, d_out):
    return pl.kernel(
        body,
        out_type=(jax.ShapeDtypeStruct((NPAD, mw), jnp.float32),
                  jax.ShapeDtypeStruct((NPAD, mw), jnp.float32)),
        mesh=plsc.VectorSubcoreMesh(core_axis_name="c",
                                    subcore_axis_name="s"),
        compiler_params=pltpu.CompilerParams(use_tc_tiling_on_sc=False),
        scratch_types=[
            pltpu.VMEM((2, kc, gw), gdtype),
            pltpu.VMEM((2, kc, _LANE), jnp.float32),
            pltpu.VMEM((2, kc, mw), jnp.float32),
            pltpu.VMEM((4, kc), jnp.int32),
            pltpu.VMEM((4, kc), jnp.int32),
        ] + [pltpu.SemaphoreType.DMA] * 10
          + [pltpu.VMEM_SHARED((NPAD, mw), jnp.float32)],
    )(ei0_2d, ei1_2d, g_in, d_in, g_out, d_out)

  return call


K2 = 200
_sc_layer1 = _make_sc_layer(160, jnp.float32, 160, _compute1, K)
_sc_layer2 = _make_sc_layer(32, jnp.float32, _LANE, _compute2, K2)


_BN = 400
_GRID = N // _BN


def _mm_body(x_ref, *refs):
  nw = len(refs) // 2
  xb = x_ref[...]
  for i in range(nw):
    y = jnp.dot(xb, refs[i][...], preferred_element_type=jnp.float32)
    refs[nw + i][...] = y.astype(refs[nw + i].dtype)


def _mm(x, ws, dtypes=None):
  din = x.shape[1]
  if dtypes is None:
    dtypes = [jnp.float32] * len(ws)
  in_specs = [pl.BlockSpec((_BN, din), lambda i: (i, 0))]
  in_specs += [pl.BlockSpec(w.shape, lambda i: (0, 0)) for w in ws]
  return pl.pallas_call(
      _mm_body,
      grid=(_GRID,),
      in_specs=in_specs,
      out_specs=[pl.BlockSpec((_BN, w.shape[1]), lambda i: (i, 0))
                 for w in ws],
      out_shape=[jax.ShapeDtypeStruct((N, w.shape[1]), dt)
                 for w, dt in zip(ws, dtypes)],
  )(x, *ws)


def _combine1_body(ai_ref, ao_ref, xl_ref, bi_ref, bo_ref, bl_ref, o_ref):
  r = lax.broadcasted_iota(jnp.int32, (12, 144), 0)
  c = lax.broadcasted_iota(jnp.int32, (12, 144), 1) // 12
  mexp = (r == c).astype(jnp.float32)

  def branch(a_ref, b_ref):
    a = a_ref[...]
    num = a[:, :144]
    den = a[:, 144:156]
    inv = 1.0 / (den + 1e-16)
    return num * jnp.dot(inv, mexp, preferred_element_type=jnp.float32) \
        + b_ref[...]

  xi = branch(ai_ref, bi_ref)
  xo = branch(ao_ref, bo_ref)
  h = 0.5 * xi + 0.5 * xo + xl_ref[...] + bl_ref[...]
  o_ref[...] = jnp.maximum(h, 0.0)


def _combine1(ai, ao, xl, bi, bo, bl):
  return pl.pallas_call(
      _combine1_body,
      grid=(_GRID,),
      in_specs=[
          pl.BlockSpec((_BN, 160), lambda i: (i, 0)),
          pl.BlockSpec((_BN, 160), lambda i: (i, 0)),
          pl.BlockSpec((_BN, 144), lambda i: (i, 0)),
          pl.BlockSpec((1, 144), lambda i: (0, 0)),
          pl.BlockSpec((1, 144), lambda i: (0, 0)),
          pl.BlockSpec((1, 144), lambda i: (0, 0)),
      ],
      out_specs=pl.BlockSpec((_BN, 144), lambda i: (i, 0)),
      out_shape=jax.ShapeDtypeStruct((N, 144), jnp.float32),
  )(ai, ao, xl, bi, bo, bl)


def _combine2_body(ai_ref, ao_ref, xl_ref, bi_ref, bo_ref, bl_ref, o_ref):
  def branch(a_ref, b_ref):
    a = a_ref[...]
    num = a[:, :10]
    den = a[:, 15:16]
    inv = 1.0 / (den + 1e-16)
    return num * inv + b_ref[...]

  xi = branch(ai_ref, bi_ref)
  xo = branch(ao_ref, bo_ref)
  o_ref[...] = 0.5 * xi + 0.5 * xo + xl_ref[...] + bl_ref[...]


def _combine2(ai, ao, xl, bi, bo, bl):
  return pl.pallas_call(
      _combine2_body,
      grid=(_GRID,),
      in_specs=[
          pl.BlockSpec((_BN, _LANE), lambda i: (i, 0)),
          pl.BlockSpec((_BN, _LANE), lambda i: (i, 0)),
          pl.BlockSpec((_BN, 10), lambda i: (i, 0)),
          pl.BlockSpec((1, 10), lambda i: (0, 0)),
          pl.BlockSpec((1, 10), lambda i: (0, 0)),
          pl.BlockSpec((1, 10), lambda i: (0, 0)),
      ],
      out_specs=pl.BlockSpec((_BN, 10), lambda i: (i, 0)),
      out_shape=jax.ShapeDtypeStruct((N, 10), jnp.float32),
  )(ai, ao, xl, bi, bo, bl)


def _fold(w, a):
  h, c = a.shape
  return jnp.einsum("dhc,hc->dh", w.reshape(w.shape[0], h, c), a)


def kernel(x, edge_index, w1i, a1si, a1di, b1i, w1o, a1so, a1do, b1o,
           lin1w, lin1b, w2i, a2si, a2di, b2i, w2o, a2so, a2do, b2o,
           lin2w, lin2b):
  ei0 = edge_index[0].reshape(E // K, K)
  ei1 = edge_index[1].reshape(E // K, K)
  ei0b = edge_index[0].reshape(E // K2, K2)
  ei1b = edge_index[1].reshape(E // K2, K2)

  z4 = jnp.zeros((128, 4), jnp.float32)
  wg_in = jnp.concatenate([w1i, _fold(w1i, a1si), z4], axis=1)
  wg_out = jnp.concatenate([w1o, _fold(w1o, a1so), z4], axis=1)
  wd_in = jnp.concatenate([_fold(w1i, a1di), z4], axis=1)
  wd_out = jnp.concatenate([_fold(w1o, a1do), z4], axis=1)

  g_in, g_out, d_in, d_out, xl1 = _mm(
      x, [wg_in, wg_out, wd_in, wd_out, lin1w])

  acc_i, acc_o = _sc_layer1(ei0, ei1, g_in, d_in, g_out, d_out)

  h1 = _combine1(acc_i, acc_o, xl1,
                 b1i.reshape(1, 144), b1o.reshape(1, 144),
                 lin1b.reshape(1, 144))

  ones16 = jnp.ones((1, _LANE), jnp.float32)
  z6 = jnp.zeros((144, 6), jnp.float32)
  wg2_in = jnp.concatenate([w2i, z6, _fold(w2i, a2si) @ ones16], axis=1)
  wg2_out = jnp.concatenate([w2o, z6, _fold(w2o, a2so) @ ones16], axis=1)
  wd2_in = _fold(w2i, a2di) @ ones16
  wd2_out = _fold(w2o, a2do) @ ones16

  g2_in, g2_out, d2_in, d2_out, xl2 = _mm(
      h1, [wg2_in, wg2_out, wd2_in, wd2_out, lin2w])

  acc2_i, acc2_o = _sc_layer2(ei0b, ei1b, g2_in, d2_in, g2_out, d2_out)

  return _combine2(acc2_i, acc2_o, xl2,
                   b2i.reshape(1, 10), b2o.reshape(1, 10),
                   lin2b.reshape(1, 10))
